# baseline (speedup 1.0000x reference)
import jax
import jax.numpy as jnp
from jax import lax
from jax.experimental import pallas as pl
from jax.experimental.pallas import tpu as pltpu

C = 768
LN = 128
SUB = 392
K = 64
CB = 32
NCHUNK = C // CB
NBUF = 4
RCHUNK = 128


def _sum_body(x_hbm, sums_ref, bufs, sems):
    j = pl.program_id(0)

    def start(chunk, slot):
        pltpu.make_async_copy(
            x_hbm.at[pl.ds(chunk * CB, CB)], bufs.at[slot], sems.at[slot]
        ).start()

    @pl.when(j == 0)
    def _prime():
        for b in range(NBUF):
            start(b, b)

    slot = lax.rem(j, NBUF)
    pltpu.make_async_copy(
        x_hbm.at[pl.ds(j * CB, CB)], bufs.at[slot], sems.at[slot]
    ).wait()
    part = jnp.sum(bufs[slot], axis=1)
    sums_ref[...] = jnp.sum(part, axis=1).reshape(1, 1, CB)

    @pl.when(j + NBUF < NCHUNK)
    def _refill():
        start(j + NBUF, slot)


def _topk_body(s_ref, idx_ref):
    totals = s_ref[...]
    vj = totals[None, :]
    jj = lax.broadcasted_iota(jnp.int32, (RCHUNK, C), 1)
    ranks = []
    for c in range(C // RCHUNK):
        vi = totals[c * RCHUNK:(c + 1) * RCHUNK][:, None]
        ii = lax.broadcasted_iota(jnp.int32, (RCHUNK, C), 0) + c * RCHUNK
        beats = (vj > vi) | ((vj == vi) & (jj < ii))
        ranks.append(jnp.sum(beats.astype(jnp.int32), axis=1))
    rank = jnp.concatenate(ranks)
    tsel = lax.broadcasted_iota(jnp.int32, (K, C), 0)
    chan = lax.broadcasted_iota(jnp.int32, (K, C), 1)
    onehot = (rank[None, :] == tsel)
    idx_ref[...] = jnp.sum(jnp.where(onehot, chan, 0), axis=1)


def kernel(input):
    x = input.reshape(C, SUB, LN)
    sums = pl.pallas_call(
        _sum_body,
        grid=(NCHUNK,),
        in_specs=[pl.BlockSpec(memory_space=pl.ANY)],
        out_specs=pl.BlockSpec((1, 1, CB), lambda j: (j, 0, 0)),
        out_shape=jax.ShapeDtypeStruct((NCHUNK, 1, CB), jnp.float32),
        scratch_shapes=[
            pltpu.VMEM((NBUF, CB, SUB, LN), jnp.float32),
            pltpu.SemaphoreType.DMA((NBUF,)),
        ],
    )(x)
    sums = sums.reshape(C)
    return pl.pallas_call(
        _topk_body,
        out_shape=jax.ShapeDtypeStruct((K,), jnp.int32),
    )(sums)


# speedup vs baseline: 3.8260x; 3.8260x over previous
import jax
import jax.numpy as jnp
from jax import lax
from jax.experimental import pallas as pl
from jax.experimental.pallas import tpu as pltpu

C = 768
R = 50176
K = 64
RB = 1024
NCHUNK = R // RB
NBUF = 6
RCHUNK = 128


def _sum_body(x_hbm, sums_ref, bufs, sems, acc_ref):
    j = pl.program_id(0)

    def start(chunk, slot):
        pltpu.make_async_copy(
            x_hbm.at[pl.ds(chunk * RB, RB)], bufs.at[slot], sems.at[slot]
        ).start()

    @pl.when(j == 0)
    def _prime():
        acc_ref[...] = jnp.zeros_like(acc_ref)
        for b in range(NBUF):
            start(b, b)

    slot = lax.rem(j, NBUF)
    pltpu.make_async_copy(
        x_hbm.at[pl.ds(j * RB, RB)], bufs.at[slot], sems.at[slot]
    ).wait()
    acc_ref[...] += jnp.sum(bufs[slot].reshape(RB // 8, 8, C), axis=0)

    @pl.when(j + NBUF < NCHUNK)
    def _refill():
        start(j + NBUF, slot)

    @pl.when(j == NCHUNK - 1)
    def _finish():
        sums_ref[...] = jnp.sum(acc_ref[...], axis=0)


def _topk_body(s_ref, idx_ref):
    totals = s_ref[...]
    vj = totals[None, :]
    jj = lax.broadcasted_iota(jnp.int32, (RCHUNK, C), 1)
    ranks = []
    for c in range(C // RCHUNK):
        vi = totals[c * RCHUNK:(c + 1) * RCHUNK][:, None]
        ii = lax.broadcasted_iota(jnp.int32, (RCHUNK, C), 0) + c * RCHUNK
        beats = (vj > vi) | ((vj == vi) & (jj < ii))
        ranks.append(jnp.sum(beats.astype(jnp.int32), axis=1))
    rank = jnp.concatenate(ranks)
    tsel = lax.broadcasted_iota(jnp.int32, (K, C), 0)
    chan = lax.broadcasted_iota(jnp.int32, (K, C), 1)
    onehot = (rank[None, :] == tsel)
    idx_ref[...] = jnp.sum(jnp.where(onehot, chan, 0), axis=1)


def kernel(input):
    x = jnp.transpose(input, (0, 2, 3, 1)).reshape(R, C)
    sums = pl.pallas_call(
        _sum_body,
        grid=(NCHUNK,),
        in_specs=[pl.BlockSpec(memory_space=pl.ANY)],
        out_specs=pl.BlockSpec((C,), lambda j: (0,)),
        out_shape=jax.ShapeDtypeStruct((C,), jnp.float32),
        scratch_shapes=[
            pltpu.VMEM((NBUF, RB, C), jnp.float32),
            pltpu.SemaphoreType.DMA((NBUF,)),
            pltpu.VMEM((8, C), jnp.float32),
        ],
    )(x)
    return pl.pallas_call(
        _topk_body,
        out_shape=jax.ShapeDtypeStruct((K,), jnp.int32),
    )(sums)


# speedup vs baseline: 3.9769x; 1.0394x over previous
import jax
import jax.numpy as jnp
from jax import lax
from jax.experimental import pallas as pl
from jax.experimental.pallas import tpu as pltpu

C = 768
R = 50176
K = 64
RB = 1024
NCHUNK = R // RB
NBUF = 6
RCHUNK = 128


def _body(x_hbm, idx_ref, bufs, sems, acc_ref):
    j = pl.program_id(0)

    def start(chunk, slot):
        pltpu.make_async_copy(
            x_hbm.at[pl.ds(chunk * RB, RB)], bufs.at[slot], sems.at[slot]
        ).start()

    @pl.when(j == 0)
    def _prime():
        acc_ref[...] = jnp.zeros_like(acc_ref)
        for b in range(NBUF):
            start(b, b)

    slot = lax.rem(j, NBUF)
    pltpu.make_async_copy(
        x_hbm.at[pl.ds(j * RB, RB)], bufs.at[slot], sems.at[slot]
    ).wait()
    acc_ref[...] += jnp.sum(bufs[slot].reshape(RB // 8, 8, C), axis=0)

    @pl.when(j + NBUF < NCHUNK)
    def _refill():
        start(j + NBUF, slot)

    @pl.when(j == NCHUNK - 1)
    def _finish():
        totals = jnp.sum(acc_ref[...], axis=0)
        vj = totals[None, :]
        jj = lax.broadcasted_iota(jnp.int32, (RCHUNK, C), 1)
        ranks = []
        for c in range(C // RCHUNK):
            vi = totals[c * RCHUNK:(c + 1) * RCHUNK][:, None]
            ii = lax.broadcasted_iota(jnp.int32, (RCHUNK, C), 0) + c * RCHUNK
            beats = (vj > vi) | ((vj == vi) & (jj < ii))
            ranks.append(jnp.sum(beats.astype(jnp.int32), axis=1))
        rank = jnp.concatenate(ranks)
        tsel = lax.broadcasted_iota(jnp.int32, (K, C), 0)
        chan = lax.broadcasted_iota(jnp.int32, (K, C), 1)
        onehot = (rank[None, :] == tsel)
        idx_ref[...] = jnp.sum(jnp.where(onehot, chan, 0), axis=1)


def kernel(input):
    x = jnp.transpose(input, (0, 2, 3, 1)).reshape(R, C)
    return pl.pallas_call(
        _body,
        grid=(NCHUNK,),
        in_specs=[pl.BlockSpec(memory_space=pl.ANY)],
        out_specs=pl.BlockSpec((K,), lambda j: (0,)),
        out_shape=jax.ShapeDtypeStruct((K,), jnp.int32),
        scratch_shapes=[
            pltpu.VMEM((NBUF, RB, C), jnp.float32),
            pltpu.SemaphoreType.DMA((NBUF,)),
            pltpu.VMEM((8, C), jnp.float32),
        ],
    )(x)
